# baseline (speedup 1.0000x reference)
import functools

import jax
import jax.numpy as jnp
from jax import lax
from jax.experimental import pallas as pl
from jax.experimental.pallas import tpu as pltpu
from jax.experimental.pallas import tpu_sc as plsc

_NN = 100000
_F = 128
_G = 512
_NC = 2
_NS = 16
_L = 16
_FH = _F // _NC
_CHUNK = 128
_NCH = (_NN + _CHUNK - 1) // _CHUNK
_CPT = (_NCH + _NS - 1) // _NS
_NCH_PAD = _CPT * _NS
_TAIL = _NN - (_NCH - 1) * _CHUNK
_TAILG = _NCH - _CPT * (_NS - 1) - 1
_DUMMY = _G
_NBUF = 10
_LAG = 5
_ACC_ROWS = _G + _NS
_NBANK = 4
_TOT_ACC = _NBANK * _ACC_ROWS
_ZROWS = _TOT_ACC // _NS
_OROWS = _G // _NS

_mesh = plsc.VectorSubcoreMesh(core_axis_name="c", subcore_axis_name="s")


@functools.partial(
    pl.kernel,
    out_type=(
        jax.ShapeDtypeStruct((_G, _F), jnp.float32),
        jax.ShapeDtypeStruct((_NN, _F), jnp.float32),
    ),
    mesh=_mesh,
    scratch_types=[
        pltpu.VMEM((_CPT, _CHUNK), jnp.int32),
        pltpu.VMEM((_NBUF, _CHUNK, _FH), jnp.float32),
        pltpu.VMEM_SHARED((_TOT_ACC, _FH), jnp.float32),
    ]
    + [pltpu.SemaphoreType.DMA] * (3 * _NBUF),
    compiler_params=pltpu.CompilerParams(use_tc_tiling_on_sc=False),
)
def _segsum(x_hbm, bp_hbm, m_hbm, xo_hbm, idx_v, rows_v, acc_sh, *sems):
    load_sems = sems[:_NBUF]
    add_sems = sems[_NBUF:2 * _NBUF]
    wb_sems = sems[2 * _NBUF:]
    cid = lax.axis_index("c")
    sid = lax.axis_index("s")
    col0 = cid * _FH
    last = _NS - 1

    zero = jnp.zeros((_L,), jnp.float32)

    def _zrow(i, carry):
        for j in range(_FH // _L):
            rows_v[0, i, pl.ds(j * _L, _L)] = zero
        return carry

    lax.fori_loop(0, _CHUNK, _zrow, 0)
    pltpu.sync_copy(
        rows_v.at[0],
        acc_sh.at[pl.ds(sid * _ZROWS, _CHUNK)],
    )
    pltpu.sync_copy(
        rows_v.at[0, pl.ds(0, _ZROWS - _CHUNK)],
        acc_sh.at[pl.ds(sid * _ZROWS + _CHUNK, _ZROWS - _CHUNK)],
    )
    pltpu.sync_copy(bp_hbm.at[pl.ds(sid * _CPT, _CPT)], idx_v)
    lane = lax.iota(jnp.int32, _L)
    bankpat = (lane % _NBANK) * _ACC_ROWS
    for g in range(_CPT):
        for j in range(_CHUNK // _L):
            idx_v[g, pl.ds(j * _L, _L)] = (
                idx_v[g, pl.ds(j * _L, _L)] + bankpat
            )

    def ranged(g, full, tail_variant):
        if g < _TAILG:
            full()
        elif g == _TAILG:
            pl.when(sid < last)(full)
            pl.when(sid == last)(tail_variant)
        else:
            pl.when(sid < last)(full)

    def issue_load(g):
        b = g % _NBUF
        base = (sid * _CPT + g) * _CHUNK

        def full():
            pltpu.async_copy(
                x_hbm.at[pl.ds(base, _CHUNK), pl.ds(col0, _FH)],
                rows_v.at[b],
                load_sems[b],
            )

        def tail():
            pltpu.async_copy(
                x_hbm.at[pl.ds((_NCH - 1) * _CHUNK, _TAIL), pl.ds(col0, _FH)],
                rows_v.at[b, pl.ds(0, _TAIL)],
                load_sems[b],
            )

        ranged(g, full, tail)

    def wait_dma(g, sem, rows_full, rows_tail):
        b = g % _NBUF

        def full():
            pltpu.make_async_copy(
                x_hbm.at[pl.ds(0, rows_full), pl.ds(0, _FH)],
                rows_v.at[b, pl.ds(0, rows_full)], sem).wait()

        def tail():
            pltpu.make_async_copy(
                x_hbm.at[pl.ds(0, rows_tail), pl.ds(0, _FH)],
                rows_v.at[b, pl.ds(0, rows_tail)], sem).wait()

        ranged(g, full, tail)

    def issue_scatter(g):
        b = g % _NBUF

        def fire():
            pltpu.async_copy(
                rows_v.at[b], acc_sh.at[idx_v.at[g]], add_sems[b], add=True
            )

        ranged(g, fire, fire)

    def issue_wb(g):
        b = g % _NBUF
        base = (sid * _CPT + g) * _CHUNK

        def full():
            pltpu.async_copy(
                rows_v.at[b],
                xo_hbm.at[pl.ds(base, _CHUNK), pl.ds(col0, _FH)],
                wb_sems[b],
            )

        def tail():
            pltpu.async_copy(
                rows_v.at[b, pl.ds(0, _TAIL)],
                xo_hbm.at[pl.ds((_NCH - 1) * _CHUNK, _TAIL),
                          pl.ds(col0, _FH)],
                wb_sems[b],
            )

        ranged(g, full, tail)

    for g in range(_LAG):
        issue_load(g)
    plsc.subcore_barrier()

    for g in range(_CPT):
        wait_dma(g, load_sems[g % _NBUF], _CHUNK, _TAIL)
        issue_scatter(g)
        issue_wb(g)
        if g >= _LAG:
            gp = g - _LAG
            wait_dma(gp, add_sems[gp % _NBUF], _CHUNK, _CHUNK)
            wait_dma(gp, wb_sems[gp % _NBUF], _CHUNK, _TAIL)
        if g + _LAG < _CPT:
            issue_load(g + _LAG)
    for g in range(_CPT - _LAG, _CPT):
        wait_dma(g, add_sems[g % _NBUF], _CHUNK, _CHUNK)
        wait_dma(g, wb_sems[g % _NBUF], _CHUNK, _TAIL)

    plsc.subcore_barrier()
    for k in range(_NBANK):
        pltpu.sync_copy(
            acc_sh.at[pl.ds(k * _ACC_ROWS + sid * _OROWS, _OROWS)],
            rows_v.at[k, pl.ds(0, _OROWS)],
        )

    def _sumrow(i, carry):
        for j in range(_FH // _L):
            sl = pl.ds(j * _L, _L)
            rows_v[_NBANK, i, sl] = (
                (rows_v[0, i, sl] + rows_v[1, i, sl])
                + (rows_v[2, i, sl] + rows_v[3, i, sl])
            )
        return carry

    lax.fori_loop(0, _OROWS, _sumrow, 0)
    pltpu.sync_copy(
        rows_v.at[_NBANK, pl.ds(0, _OROWS)],
        m_hbm.at[pl.ds(sid * _OROWS, _OROWS), pl.ds(col0, _FH)],
    )


def kernel(x, edge_index, batch):
    pad = jnp.full((_NCH_PAD * _CHUNK - _NN,), _DUMMY, jnp.int32)
    batch_p = jnp.concatenate([batch, pad]).reshape(_NCH_PAD, _CHUNK)
    m, x_out = _segsum(x, batch_p)
    return (m, x_out)


# speedup vs baseline: 1.0448x; 1.0448x over previous
import functools

import jax
import jax.numpy as jnp
from jax import lax
from jax.experimental import pallas as pl
from jax.experimental.pallas import tpu as pltpu
from jax.experimental.pallas import tpu_sc as plsc

_NN = 100000
_F = 128
_G = 512
_NC = 2
_NS = 16
_L = 16
_FH = _F // _NC
_CHUNK = 128
_NCH = (_NN + _CHUNK - 1) // _CHUNK
_CPT = (_NCH + _NS - 1) // _NS
_NCH_PAD = _CPT * _NS
_TAIL = _NN - (_NCH - 1) * _CHUNK
_TAILG = _NCH - _CPT * (_NS - 1) - 1
_DUMMY = _G
_NBUF = 6
_LAG = 3
_ACC_ROWS = _G + _NS
_ZROWS = _ACC_ROWS // _NS
_OROWS = _G // _NS

_mesh = plsc.VectorSubcoreMesh(core_axis_name="c", subcore_axis_name="s")


@functools.partial(
    pl.kernel,
    out_type=(
        jax.ShapeDtypeStruct((_G, _F), jnp.float32),
        jax.ShapeDtypeStruct((_NN, _F), jnp.float32),
    ),
    mesh=_mesh,
    scratch_types=[
        pltpu.VMEM((_CPT, _CHUNK), jnp.int32),
        pltpu.VMEM((_NBUF, _CHUNK, _FH), jnp.float32),
        pltpu.VMEM_SHARED((_ACC_ROWS, _FH), jnp.float32),
    ]
    + [pltpu.SemaphoreType.DMA] * (3 * _NBUF),
    compiler_params=pltpu.CompilerParams(use_tc_tiling_on_sc=False),
)
def _segsum(x_hbm, bp_hbm, m_hbm, xo_hbm, idx_v, rows_v, acc_sh, *sems):
    load_sems = sems[:_NBUF]
    add_sems = sems[_NBUF:2 * _NBUF]
    wb_sems = sems[2 * _NBUF:]
    cid = lax.axis_index("c")
    sid = lax.axis_index("s")
    col0 = cid * _FH
    last = _NS - 1

    zero = jnp.zeros((_L,), jnp.float32)
    for i in range(_ZROWS):
        for j in range(_FH // _L):
            rows_v[0, i, pl.ds(j * _L, _L)] = zero
    pltpu.sync_copy(
        rows_v.at[0, pl.ds(0, _ZROWS)],
        acc_sh.at[pl.ds(sid * _ZROWS, _ZROWS)],
    )
    pltpu.sync_copy(bp_hbm.at[pl.ds(sid * _CPT, _CPT)], idx_v)

    def ranged(g, full, tail_variant):
        if g < _TAILG:
            full()
        elif g == _TAILG:
            pl.when(sid < last)(full)
            pl.when(sid == last)(tail_variant)
        else:
            pl.when(sid < last)(full)

    def issue_load(g):
        b = g % _NBUF
        base = (sid * _CPT + g) * _CHUNK

        def full():
            pltpu.async_copy(
                x_hbm.at[pl.ds(base, _CHUNK), pl.ds(col0, _FH)],
                rows_v.at[b],
                load_sems[b],
            )

        def tail():
            pltpu.async_copy(
                x_hbm.at[pl.ds((_NCH - 1) * _CHUNK, _TAIL), pl.ds(col0, _FH)],
                rows_v.at[b, pl.ds(0, _TAIL)],
                load_sems[b],
            )

        ranged(g, full, tail)

    def wait_dma(g, sem, rows_full, rows_tail):
        b = g % _NBUF

        def full():
            pltpu.make_async_copy(
                x_hbm.at[pl.ds(0, rows_full), pl.ds(0, _FH)],
                rows_v.at[b, pl.ds(0, rows_full)], sem).wait()

        def tail():
            pltpu.make_async_copy(
                x_hbm.at[pl.ds(0, rows_tail), pl.ds(0, _FH)],
                rows_v.at[b, pl.ds(0, rows_tail)], sem).wait()

        ranged(g, full, tail)

    def issue_scatter(g):
        b = g % _NBUF

        def fire():
            pltpu.async_copy(
                rows_v.at[b], acc_sh.at[idx_v.at[g]], add_sems[b], add=True
            )

        ranged(g, fire, fire)

    def issue_wb(g):
        b = g % _NBUF
        base = (sid * _CPT + g) * _CHUNK

        def full():
            pltpu.async_copy(
                rows_v.at[b],
                xo_hbm.at[pl.ds(base, _CHUNK), pl.ds(col0, _FH)],
                wb_sems[b],
            )

        def tail():
            pltpu.async_copy(
                rows_v.at[b, pl.ds(0, _TAIL)],
                xo_hbm.at[pl.ds((_NCH - 1) * _CHUNK, _TAIL),
                          pl.ds(col0, _FH)],
                wb_sems[b],
            )

        ranged(g, full, tail)

    for g in range(_LAG):
        issue_load(g)
    plsc.subcore_barrier()

    for g in range(_CPT):
        wait_dma(g, load_sems[g % _NBUF], _CHUNK, _TAIL)
        issue_scatter(g)
        issue_wb(g)
        if g >= _LAG:
            gp = g - _LAG
            wait_dma(gp, add_sems[gp % _NBUF], _CHUNK, _CHUNK)
            wait_dma(gp, wb_sems[gp % _NBUF], _CHUNK, _TAIL)
        if g + _LAG < _CPT:
            issue_load(g + _LAG)
    for g in range(_CPT - _LAG, _CPT):
        wait_dma(g, add_sems[g % _NBUF], _CHUNK, _CHUNK)
        wait_dma(g, wb_sems[g % _NBUF], _CHUNK, _TAIL)

    plsc.subcore_barrier()
    pltpu.sync_copy(
        acc_sh.at[pl.ds(sid * _OROWS, _OROWS)],
        m_hbm.at[pl.ds(sid * _OROWS, _OROWS), pl.ds(col0, _FH)],
    )


def kernel(x, edge_index, batch):
    pad = jnp.full((_NCH_PAD * _CHUNK - _NN,), _DUMMY, jnp.int32)
    batch_p = jnp.concatenate([batch, pad]).reshape(_NCH_PAD, _CHUNK)
    m, x_out = _segsum(x, batch_p)
    return (m, x_out)
